# baseline (speedup 1.0000x reference)
import dataclasses
import functools
import jax
import jax.numpy as jnp
from jax import lax
from jax.experimental import pallas as pl
from jax.experimental.pallas import tpu as pltpu
from jax.experimental.pallas import tpu_sc as plsc

NUM_EXPERTS = 16
HID = 1024
B_, C_, T_ = 2, 1024, 4096
TCB = 512
NT = T_ // TCB
LANES = 128
KSUB = TCB // LANES
NCHUNK = HID // 16


def _reduce_body(x_ref, sums_ref, acc_ref):
    j = pl.program_id(0)
    xb = x_ref[...]
    s = xb[:, :, 0 * LANES:1 * LANES]
    for k in range(1, KSUB):
        s = s + xb[:, :, k * LANES:(k + 1) * LANES]

    @pl.when(j == 0)
    def _():
        acc_ref[...] = s

    @pl.when(j > 0)
    def _():
        acc_ref[...] = acc_ref[...] + s

    @pl.when(j == NT - 1)
    def _():
        sums_ref[...] = jnp.sum(acc_ref[...], axis=-1)


def _tc_reduce(xs):
    return pl.pallas_call(
        _reduce_body,
        grid=(NT,),
        in_specs=[pl.BlockSpec((B_, C_, TCB), lambda j: (0, 0, j))],
        out_specs=pl.BlockSpec((B_, C_), lambda j: (0, 0)),
        out_shape=jax.ShapeDtypeStruct((B_, C_), jnp.float32),
        scratch_shapes=[pltpu.VMEM((B_, C_, LANES), jnp.float32)],
        compiler_params=pltpu.CompilerParams(
            dimension_semantics=("arbitrary",)),
    )(xs)


def _sc_gate(sums, gate_w, gate_b, gammas, betas):
    mesh = plsc.VectorSubcoreMesh(core_axis_name="c", subcore_axis_name="s")
    cp = pltpu.CompilerParams()
    if "needs_layout_passes" in pltpu.CompilerParams.__dataclass_fields__:
        cp = dataclasses.replace(cp, needs_layout_passes=False)

    @functools.partial(
        pl.kernel,
        compiler_params=cp,
        out_type=[
            jax.ShapeDtypeStruct((B_, NUM_EXPERTS), jnp.float32),
            jax.ShapeDtypeStruct((B_, NUM_EXPERTS), jnp.float32),
        ],
        mesh=mesh,
        scratch_types=[
            pltpu.VMEM((HID,), jnp.float32),
            pltpu.VMEM((HID,), jnp.float32),
            pltpu.VMEM((NUM_EXPERTS,), jnp.float32),
            pltpu.VMEM((NUM_EXPERTS, NUM_EXPERTS), jnp.float32),
            pltpu.VMEM((NUM_EXPERTS,), jnp.float32),
            pltpu.VMEM((NUM_EXPERTS,), jnp.float32),
            pltpu.VMEM((NUM_EXPERTS,), jnp.float32),
            pltpu.VMEM_SHARED((NUM_EXPERTS, NUM_EXPERTS), jnp.float32),
            pltpu.SemaphoreType.DMA,
        ],
    )
    def gate_kernel(sums_hbm, gw_hbm, gb_hbm, gam_hbm, bet_hbm,
                    gout_hbm, bout_hbm,
                    gwrow_v, sums_v, tmp_v, grid_v, gb_v, gam_v, bet_v,
                    shared, sem):
        cid = lax.axis_index("c")
        e = lax.axis_index("s")

        @pl.when(cid == 0)
        def _():
            pltpu.async_copy(gw_hbm.at[e], gwrow_v, sem).wait()
            for b in range(B_):
                pltpu.async_copy(sums_hbm.at[b], sums_v, sem).wait()

                def step(k, acc):
                    return acc + (gwrow_v[pl.ds(k * 16, 16)]
                                  * sums_v[pl.ds(k * 16, 16)])

                acc = lax.fori_loop(0, NCHUNK, step,
                                    jnp.zeros((NUM_EXPERTS,), jnp.float32))
                score = jnp.sum(acc)
                tmp_v[...] = jnp.full((NUM_EXPERTS,), score, jnp.float32)
                pltpu.sync_copy(tmp_v, shared.at[e])
                plsc.subcore_barrier()

                @pl.when(e == 0)
                def _():
                    pltpu.async_copy(gb_hbm, gb_v, sem).wait()
                    pltpu.async_copy(gam_hbm, gam_v, sem).wait()
                    pltpu.async_copy(bet_hbm, bet_v, sem).wait()
                    pltpu.sync_copy(shared, grid_v)
                    iota = lax.iota(jnp.int32, NUM_EXPERTS)
                    scores = jnp.zeros((NUM_EXPERTS,), jnp.float32)
                    for ee in range(NUM_EXPERTS):
                        scores = scores + jnp.where(
                            iota == ee, grid_v[ee], 0.0)
                    scores = scores * (1.0 / T_) + gb_v[...]
                    m = jnp.max(scores)
                    idx = jnp.min(
                        jnp.where(scores >= m, iota, NUM_EXPERTS))
                    sel = iota == idx
                    tmp_v[...] = jnp.where(sel, gam_v[...], 0.0)
                    pltpu.sync_copy(tmp_v, gout_hbm.at[b])
                    tmp_v[...] = jnp.where(sel, bet_v[...], 0.0)
                    pltpu.sync_copy(tmp_v, bout_hbm.at[b])

                plsc.subcore_barrier()

    return gate_kernel(sums, gate_w, gate_b, gammas, betas)


def _apply_body(x_ref, gsel_ref, bsel_ref, out_ref):
    g = jnp.sum(gsel_ref[...], axis=-1)[:, None, None]
    b = jnp.sum(bsel_ref[...], axis=-1)[:, None, None]
    out_ref[...] = x_ref[...] * g + b


def _tc_apply(xs, gsel, bsel):
    return pl.pallas_call(
        _apply_body,
        grid=(NT,),
        in_specs=[
            pl.BlockSpec((B_, C_, TCB), lambda j: (0, 0, j)),
            pl.BlockSpec((B_, NUM_EXPERTS), lambda j: (0, 0)),
            pl.BlockSpec((B_, NUM_EXPERTS), lambda j: (0, 0)),
        ],
        out_specs=pl.BlockSpec((B_, C_, TCB), lambda j: (0, 0, j)),
        out_shape=jax.ShapeDtypeStruct((B_, C_, T_), jnp.float32),
        compiler_params=pltpu.CompilerParams(
            dimension_semantics=("arbitrary",)),
    )(xs, gsel, bsel)


def kernel(x, gate_w, gate_b, gammas, betas):
    xs = x.reshape(B_, C_, T_)
    sums = _tc_reduce(xs)
    gsel, bsel = _sc_gate(sums, gate_w, gate_b, gammas, betas)
    out = _tc_apply(xs, gsel, bsel)
    return out.reshape(B_, C_, T_, 1)


# speedup vs baseline: 4.0430x; 4.0430x over previous
import jax
import jax.numpy as jnp
from jax.experimental import pallas as pl
from jax.experimental.pallas import tpu as pltpu

NUM_EXPERTS = 16
HID = 1024
B_, C_, T_ = 2, 1024, 4096
LANES = 128
TH = T_ // LANES
RTOT = B_ * C_
RB = 256
NT = RTOT // RB
NB0 = C_ // RB


def _body(x_ref, gw_ref, gb_ref, gam_ref, bet_ref, out_ref,
          stash_ref, acc_ref, gsel_ref, bsel_ref):
    p = pl.program_id(0)
    j = pl.program_id(1)

    @pl.when(p == 0)
    def _reduce():
        xb = x_ref[...]
        stash_ref[pl.ds(j * RB, RB)] = xb
        s = xb[:, 0:8, :]
        for k in range(1, TH // 8):
            s = s + xb[:, 8 * k:8 * (k + 1), :]
        acc_ref[pl.ds(j * RB, RB)] = jnp.sum(s, axis=1)

        @pl.when(j == NT - 1)
        def _gate():
            gi = acc_ref[...]
            iota = jax.lax.broadcasted_iota(
                jnp.int32, (NUM_EXPERTS, 1), 0)
            for b in range(B_):
                pb = jax.lax.dot_general(
                    gw_ref[...], gi[b * C_:(b + 1) * C_, :],
                    (((1,), (0,)), ((), ())),
                    preferred_element_type=jnp.float32,
                )
                scores = (jnp.sum(pb, axis=-1, keepdims=True) * (1.0 / T_)
                          + gb_ref[...])
                m = jnp.max(scores, axis=0, keepdims=True)
                idx = jnp.min(
                    jnp.where(scores >= m, iota, NUM_EXPERTS),
                    axis=0, keepdims=True)
                sel = iota == idx
                gsel_ref[b:b + 1, :] = jnp.sum(
                    jnp.where(sel, gam_ref[...], 0.0), axis=0, keepdims=True)
                bsel_ref[b:b + 1, :] = jnp.sum(
                    jnp.where(sel, bet_ref[...], 0.0), axis=0, keepdims=True)

    @pl.when(p == 1)
    def _apply():
        gsel = gsel_ref[...]
        bsel = bsel_ref[...]
        g = jnp.where(j < NB0, gsel[0:1, 0:1], gsel[1:2, 0:1])
        b = jnp.where(j < NB0, bsel[0:1, 0:1], bsel[1:2, 0:1])
        out_ref[...] = (stash_ref[pl.ds(j * RB, RB)] * g[:, :, None]
                        + b[:, :, None])


def kernel(x, gate_w, gate_b, gammas, betas):
    xs = x.reshape(RTOT, TH, LANES)
    out = pl.pallas_call(
        _body,
        grid=(2, NT),
        in_specs=[
            pl.BlockSpec((RB, TH, LANES),
                         lambda p, j: (jnp.where(p == 0, j, NT - 1), 0, 0)),
            pl.BlockSpec((NUM_EXPERTS, HID), lambda p, j: (0, 0)),
            pl.BlockSpec((NUM_EXPERTS, 1), lambda p, j: (0, 0)),
            pl.BlockSpec((NUM_EXPERTS, 1), lambda p, j: (0, 0)),
            pl.BlockSpec((NUM_EXPERTS, 1), lambda p, j: (0, 0)),
        ],
        out_specs=pl.BlockSpec(
            (RB, TH, LANES), lambda p, j: (jnp.where(p == 0, 0, j), 0, 0)),
        out_shape=jax.ShapeDtypeStruct((RTOT, TH, LANES), jnp.float32),
        scratch_shapes=[
            pltpu.VMEM((RTOT, TH, LANES), jnp.float32),
            pltpu.VMEM((RTOT, LANES), jnp.float32),
            pltpu.VMEM((B_, 1), jnp.float32),
            pltpu.VMEM((B_, 1), jnp.float32),
        ],
        compiler_params=pltpu.CompilerParams(
            dimension_semantics=("arbitrary", "arbitrary")),
    )(xs, gate_w, gate_b.reshape(NUM_EXPERTS, 1),
      gammas.reshape(NUM_EXPERTS, 1), betas.reshape(NUM_EXPERTS, 1))
    return out.reshape(B_, C_, T_, 1)
